# Initial kernel scaffold; baseline (speedup 1.0000x reference)
import jax
import jax.numpy as jnp
from jax.experimental import pallas as pl


def kernel(h, edge_index):
    raise NotImplementedError("write your pallas kernel here")


# speedup vs baseline: 3.6297x; 3.6297x over previous
import functools

import jax
import jax.numpy as jnp
from jax import lax
from jax.experimental import pallas as pl
from jax.experimental.pallas import tpu as pltpu
from jax.experimental.pallas import tpu_sc as plsc

NC = 2
NS = 16
NW = NC * NS
LANES = 16

_PERM_DNUMS = lax.GatherDimensionNumbers(
    offset_dims=(), collapsed_slice_dims=(0,), start_index_map=(0,))


def _lane_permute(x, idx):
    return lax.gather(
        x, idx[:, None], dimension_numbers=_PERM_DNUMS, slice_sizes=(1,),
        mode=lax.GatherScatterMode.PROMISE_IN_BOUNDS)


@functools.partial(jax.jit, static_argnames=("epw", "chunk"))
def _sc_edge_dot(h, src, dst, *, epw, chunk):
    e_total = src.shape[0]
    d_feat = h.shape[1]
    n_chunks = epw // chunk
    mesh = plsc.VectorSubcoreMesh(
        core_axis_name="c", subcore_axis_name="s",
        num_cores=NC, num_subcores=NS)

    @functools.partial(
        pl.kernel,
        out_type=jax.ShapeDtypeStruct((e_total,), jnp.float32),
        mesh=mesh,
        scratch_types=[
            pltpu.VMEM((epw,), jnp.int32),
            pltpu.VMEM((epw,), jnp.int32),
            pltpu.VMEM((chunk, d_feat), jnp.float32),
            pltpu.VMEM((chunk, d_feat), jnp.float32),
            pltpu.VMEM((epw,), jnp.float32),
            pltpu.SemaphoreType.DMA,
            pltpu.SemaphoreType.DMA,
        ],
    )
    def body(h_hbm, src_hbm, dst_hbm, out_hbm,
             sidx, didx, ubuf, vbuf, sbuf, sem_u, sem_v):
        wid = lax.axis_index("s") * NC + lax.axis_index("c")
        base = wid * epw
        pltpu.sync_copy(src_hbm.at[pl.ds(base, epw)], sidx)
        pltpu.sync_copy(dst_hbm.at[pl.ds(base, epw)], didx)

        def chunk_body(i, carry):
            off = i * chunk
            cp_u = pltpu.async_copy(
                h_hbm.at[sidx.at[pl.ds(off, chunk)]], ubuf, sem_u)
            cp_v = pltpu.async_copy(
                h_hbm.at[didx.at[pl.ds(off, chunk)]], vbuf, sem_v)
            cp_u.wait()
            cp_v.wait()

            lane = lax.iota(jnp.int32, LANES)

            def group_body(g, c2):
                res = jnp.zeros((LANES,), jnp.float32)
                for j in range(LANES):
                    e = g * LANES + j
                    acc = ubuf[e, pl.ds(0, LANES)] * vbuf[e, pl.ds(0, LANES)]
                    for k in range(1, d_feat // LANES):
                        acc = acc + (ubuf[e, pl.ds(k * LANES, LANES)]
                                     * vbuf[e, pl.ds(k * LANES, LANES)])
                    for sh in (8, 4, 2, 1):
                        acc = acc + _lane_permute(acc, lane ^ sh)
                    res = jnp.where(lane == j, acc, res)
                sbuf[pl.ds(off + g * LANES, LANES)] = res
                return c2

            return lax.fori_loop(0, chunk // LANES, group_body, carry)

        lax.fori_loop(0, n_chunks, chunk_body, 0)
        pltpu.sync_copy(sbuf, out_hbm.at[pl.ds(base, epw)])

    return body(h, src, dst)


def kernel(h, edge_index):
    e_total = edge_index.shape[1]
    assert e_total % NW == 0
    epw = e_total // NW
    chunk = 80
    assert epw % chunk == 0 and chunk % 8 == 0
    ei = edge_index.astype(jnp.int32)
    score = _sc_edge_dot(h, ei[0], ei[1], epw=epw, chunk=chunk)
    return score[:, None]


# speedup vs baseline: 9.5505x; 2.6312x over previous
import functools

import jax
import jax.numpy as jnp
from jax import lax
from jax.experimental import pallas as pl
from jax.experimental.pallas import tpu as pltpu
from jax.experimental.pallas import tpu_sc as plsc

NC = 2
NS = 16
NW = NC * NS
LANES = 16

_PERM_DNUMS = lax.GatherDimensionNumbers(
    offset_dims=(), collapsed_slice_dims=(0,), start_index_map=(0,))


def _lane_permute(x, idx):
    return lax.gather(
        x, idx[:, None], dimension_numbers=_PERM_DNUMS, slice_sizes=(1,),
        mode=lax.GatherScatterMode.PROMISE_IN_BOUNDS)


@functools.partial(jax.jit, static_argnames=("epw", "chunk"))
def _sc_edge_dot(h, src, dst, *, epw, chunk):
    e_total = src.shape[0]
    d_words = h.shape[1]
    n_chunks = epw // chunk
    assert n_chunks % 2 == 1
    n_pairs = n_chunks // 2
    mesh = plsc.VectorSubcoreMesh(
        core_axis_name="c", subcore_axis_name="s",
        num_cores=NC, num_subcores=NS)

    @functools.partial(
        pl.kernel,
        out_type=jax.ShapeDtypeStruct((e_total,), jnp.float32),
        mesh=mesh,
        scratch_types=[
            pltpu.VMEM((epw,), jnp.int32),
            pltpu.VMEM((epw,), jnp.int32),
            pltpu.VMEM((chunk, d_words), jnp.int32),
            pltpu.VMEM((chunk, d_words), jnp.int32),
            pltpu.VMEM((chunk, d_words), jnp.int32),
            pltpu.VMEM((chunk, d_words), jnp.int32),
            pltpu.VMEM((epw,), jnp.float32),
            pltpu.SemaphoreType.DMA,
            pltpu.SemaphoreType.DMA,
            pltpu.SemaphoreType.DMA,
            pltpu.SemaphoreType.DMA,
        ],
        compiler_params=pltpu.CompilerParams(
            needs_layout_passes=False, use_tc_tiling_on_sc=False),
    )
    def body(h_hbm, src_hbm, dst_hbm, out_hbm,
             sidx, didx, ub0, vb0, ub1, vb1, sbuf,
             sem_u0, sem_v0, sem_u1, sem_v1):
        wid = lax.axis_index("s") * NC + lax.axis_index("c")
        base = wid * epw
        pltpu.sync_copy(src_hbm.at[pl.ds(base, epw)], sidx)
        pltpu.sync_copy(dst_hbm.at[pl.ds(base, epw)], didx)

        lane = lax.iota(jnp.int32, LANES)

        def gather_start(i, ub, vb, sem_u, sem_v):
            off = i * chunk
            pltpu.async_copy(h_hbm.at[sidx.at[pl.ds(off, chunk)]], ub, sem_u)
            pltpu.async_copy(h_hbm.at[didx.at[pl.ds(off, chunk)]], vb, sem_v)

        def gather_wait(i, ub, vb, sem_u, sem_v):
            off = i * chunk
            pltpu.make_async_copy(
                h_hbm.at[sidx.at[pl.ds(off, chunk)]], ub, sem_u).wait()
            pltpu.make_async_copy(
                h_hbm.at[didx.at[pl.ds(off, chunk)]], vb, sem_v).wait()

        def compute(i, ub, vb):
            off = i * chunk

            def group_body(g, c2):
                res = jnp.zeros((LANES,), jnp.float32)
                for j in range(LANES):
                    e = g * LANES + j
                    acc = None
                    for k in range(d_words // LANES):
                        uw = plsc.bitcast(
                            ub[e, pl.ds(k * LANES, LANES)], jnp.bfloat16)
                        vw = plsc.bitcast(
                            vb[e, pl.ds(k * LANES, LANES)], jnp.bfloat16)
                        ua, uo = plsc.unpack(
                            uw, format=plsc.PackFormat.INTERLEAVED,
                            preferred_element_type=jnp.float32)
                        va, vo = plsc.unpack(
                            vw, format=plsc.PackFormat.INTERLEAVED,
                            preferred_element_type=jnp.float32)
                        t = ua * va + uo * vo
                        acc = t if acc is None else acc + t
                    for sh in (8, 4, 2, 1):
                        acc = acc + _lane_permute(acc, lane ^ sh)
                    res = jnp.where(lane == j, acc, res)
                sbuf[pl.ds(off + g * LANES, LANES)] = res
                return c2

            lax.fori_loop(0, chunk // LANES, group_body, 0)

        gather_start(0, ub0, vb0, sem_u0, sem_v0)

        def pair_body(p, carry):
            i0 = 2 * p
            gather_start(i0 + 1, ub1, vb1, sem_u1, sem_v1)
            gather_wait(i0, ub0, vb0, sem_u0, sem_v0)
            compute(i0, ub0, vb0)
            gather_start(i0 + 2, ub0, vb0, sem_u0, sem_v0)
            gather_wait(i0 + 1, ub1, vb1, sem_u1, sem_v1)
            compute(i0 + 1, ub1, vb1)
            return carry

        lax.fori_loop(0, n_pairs, pair_body, 0)
        gather_wait(n_chunks - 1, ub0, vb0, sem_u0, sem_v0)
        compute(n_chunks - 1, ub0, vb0)

        pltpu.sync_copy(sbuf, out_hbm.at[pl.ds(base, epw)])

    return body(h, src, dst)


def kernel(h, edge_index):
    e_total = edge_index.shape[1]
    assert e_total % NW == 0
    epw = e_total // NW
    chunk = 80
    assert epw % chunk == 0 and chunk % LANES == 0
    ei = edge_index.astype(jnp.int32)
    d_feat = h.shape[1]
    h_packed = lax.bitcast_convert_type(
        h.astype(jnp.bfloat16).reshape(h.shape[0], d_feat // 2, 2), jnp.int32)
    score = _sc_edge_dot(h_packed, ei[0], ei[1], epw=epw, chunk=chunk)
    return score[:, None]
